# initial kernel scaffold (unmeasured)
import jax
import jax.numpy as jnp
from jax import lax
from jax.experimental import pallas as pl
from jax.experimental.pallas import tpu as pltpu

N_DEV = 4
SQ = 2048
SKV = 2048
HQ = 8
DH = 128
DM = 1024
SCALE = 0.08838834764831843
QBLK = 512
MASK_BLK = 64

_MESH = pl.DeviceIdType.MESH


def _scatter_kv(k_ext, v_ext):

    def body(k_hbm, v_hbm, kout, vout, local_sems, send_sems, recv_sems,
             credit_sem, ack_sem):
        my = lax.axis_index("i")
        barrier = pltpu.get_barrier_semaphore()

        @pl.when(my == 0)
        def _():
            for p in (1, 2, 3):
                pl.semaphore_signal(barrier, inc=1, device_id=(p,),
                                    device_id_type=_MESH)
            pl.semaphore_wait(barrier, 3)

            ck = pltpu.make_async_copy(
                k_hbm.at[0, :, pl.ds(0, HQ), :], kout, local_sems.at[0])
            cv = pltpu.make_async_copy(
                v_hbm.at[0, :, pl.ds(0, HQ), :], vout, local_sems.at[1])
            ck.start()
            cv.start()

            rdmas = []
            for p in (1, 2, 3):
                rk = pltpu.make_async_remote_copy(
                    src_ref=k_hbm.at[0, :, pl.ds(HQ * p, HQ), :],
                    dst_ref=kout,
                    send_sem=send_sems.at[2 * (p - 1)],
                    recv_sem=recv_sems.at[0],
                    device_id=(p,), device_id_type=_MESH)
                rv = pltpu.make_async_remote_copy(
                    src_ref=v_hbm.at[0, :, pl.ds(HQ * p, HQ), :],
                    dst_ref=vout,
                    send_sem=send_sems.at[2 * (p - 1) + 1],
                    recv_sem=recv_sems.at[1],
                    device_id=(p,), device_id_type=_MESH)
                rk.start()
                rv.start()
                rdmas.append(rk)
                rdmas.append(rv)

            ck.wait()
            cv.wait()
            for r in rdmas:
                r.wait_send()
            pl.semaphore_wait(credit_sem, 3)
            for p in (1, 2, 3):
                pl.semaphore_signal(ack_sem, inc=1, device_id=(p,),
                                    device_id_type=_MESH)

        @pl.when(my != 0)
        def _():
            pl.semaphore_signal(barrier, inc=1, device_id=(0,),
                                device_id_type=_MESH)
            pl.semaphore_wait(barrier, 1)
            rk = pltpu.make_async_remote_copy(
                src_ref=kout, dst_ref=kout,
                send_sem=send_sems.at[0], recv_sem=recv_sems.at[0],
                device_id=(0,), device_id_type=_MESH)
            rv = pltpu.make_async_remote_copy(
                src_ref=vout, dst_ref=vout,
                send_sem=send_sems.at[1], recv_sem=recv_sems.at[1],
                device_id=(0,), device_id_type=_MESH)
            rk.wait_recv()
            rv.wait_recv()
            pl.semaphore_signal(credit_sem, inc=1, device_id=(0,),
                                device_id_type=_MESH)
            pl.semaphore_wait(ack_sem, 1)

    return pl.pallas_call(
        body,
        out_shape=[
            jax.ShapeDtypeStruct((SKV, HQ, DH), jnp.float32),
            jax.ShapeDtypeStruct((SKV, HQ, DH), jnp.float32),
        ],
        in_specs=[
            pl.BlockSpec(memory_space=pltpu.ANY),
            pl.BlockSpec(memory_space=pltpu.ANY),
        ],
        out_specs=[
            pl.BlockSpec(memory_space=pltpu.VMEM),
            pl.BlockSpec(memory_space=pltpu.VMEM),
        ],
        scratch_shapes=[
            pltpu.SemaphoreType.DMA((2,)),
            pltpu.SemaphoreType.DMA((6,)),
            pltpu.SemaphoreType.DMA((2,)),
            pltpu.SemaphoreType.REGULAR,
            pltpu.SemaphoreType.REGULAR,
        ],
        compiler_params=pltpu.CompilerParams(collective_id=0),
    )(k_ext, v_ext)


def _qproj(x, wq):
    def body(x_ref, wq_ref, q_ref):
        q_ref[:, :] = jnp.dot(x_ref[0], wq_ref[:, :],
                              preferred_element_type=jnp.float32)

    return pl.pallas_call(
        body,
        out_shape=jax.ShapeDtypeStruct((SQ, DM), jnp.float32),
        in_specs=[
            pl.BlockSpec(memory_space=pltpu.VMEM),
            pl.BlockSpec(memory_space=pltpu.VMEM),
        ],
        out_specs=pl.BlockSpec(memory_space=pltpu.VMEM),
    )(x, wq)


def _attn(q, kh, vh):

    def body(q_ref, k_ref, v_ref, o_ref):
        qi = pl.program_id(1)
        s = lax.dot_general(
            q_ref[:, :], k_ref[:, 0, :],
            (((1,), (1,)), ((), ())),
            preferred_element_type=jnp.float32) * SCALE
        r = qi * QBLK + lax.broadcasted_iota(jnp.int32, (QBLK, SKV), 0)
        c = lax.broadcasted_iota(jnp.int32, (QBLK, SKV), 1)
        s = jnp.where((c // MASK_BLK) <= (r // MASK_BLK), s, -1e9)
        m = jnp.max(s, axis=1, keepdims=True)
        w = jnp.exp(s - m)
        w = w / jnp.sum(w, axis=1, keepdims=True)
        o_ref[:, :] = jnp.dot(w, v_ref[:, 0, :],
                              preferred_element_type=jnp.float32)

    return pl.pallas_call(
        body,
        grid=(HQ, SQ // QBLK),
        in_specs=[
            pl.BlockSpec((QBLK, DH), lambda h, qi: (qi, h)),
            pl.BlockSpec((SKV, 1, DH), lambda h, qi: (0, h, 0)),
            pl.BlockSpec((SKV, 1, DH), lambda h, qi: (0, h, 0)),
        ],
        out_specs=pl.BlockSpec((QBLK, DH), lambda h, qi: (qi, h)),
        out_shape=jax.ShapeDtypeStruct((SQ, DM), jnp.float32),
    )(q, kh, vh)


def _wo_allreduce(ctx, wo):

    def body(ctx_ref, wo_ref, out_ref, comm, send_sems, recv_sems):
        my = lax.axis_index("i")
        left = (my + N_DEV - 1) % N_DEV
        right = (my + 1) % N_DEV

        barrier = pltpu.get_barrier_semaphore()
        for nbr in (left, right):
            pl.semaphore_signal(barrier, inc=1, device_id=(nbr,),
                                device_id_type=_MESH)
        pl.semaphore_wait(barrier, 2)

        partial = jnp.dot(ctx_ref[:, :], wo_ref[:, :],
                          preferred_element_type=jnp.float32)
        out_ref[0] = partial
        comm[0] = partial

        for h in range(N_DEV - 1):
            send_slot = h % 2
            recv_slot = (h + 1) % 2
            rdma = pltpu.make_async_remote_copy(
                src_ref=comm.at[send_slot],
                dst_ref=comm.at[recv_slot],
                send_sem=send_sems.at[send_slot],
                recv_sem=recv_sems.at[recv_slot],
                device_id=(right,), device_id_type=_MESH)
            rdma.start()
            rdma.wait()
            out_ref[0] += comm[recv_slot]

    return pl.pallas_call(
        body,
        out_shape=jax.ShapeDtypeStruct((1, SQ, DM), jnp.float32),
        in_specs=[
            pl.BlockSpec(memory_space=pltpu.VMEM),
            pl.BlockSpec(memory_space=pltpu.VMEM),
        ],
        out_specs=pl.BlockSpec(memory_space=pltpu.VMEM),
        scratch_shapes=[
            pltpu.VMEM((2, SQ, DM), jnp.float32),
            pltpu.SemaphoreType.DMA((2,)),
            pltpu.SemaphoreType.DMA((2,)),
        ],
        compiler_params=pltpu.CompilerParams(collective_id=1),
    )(ctx, wo)


def kernel(x, Wq, K_ext, V_ext, Wo):
    kh, vh = _scatter_kv(K_ext, V_ext)
    q = _qproj(x, Wq)
    ctx = _attn(q, kh, vh)
    return _wo_allreduce(ctx, Wo)


# baseline (device time: 777131 ns/iter reference)
import jax
import jax.numpy as jnp
from jax import lax
from jax.experimental import pallas as pl
from jax.experimental.pallas import tpu as pltpu

N_DEV = 4
SQ = 2048
SKV = 2048
HQ = 8
DH = 128
DM = 1024
SCALE = 0.08838834764831843
QBLK = 512
MASK_BLK = 64

_MESH = pl.DeviceIdType.MESH


def _scatter_kv(k_ext, v_ext):

    def body(k_hbm, v_hbm, kout, vout, local_sems, send_sems, recv_sems,
             credit_sem, ack_sem):
        my = lax.axis_index("i")
        barrier = pltpu.get_barrier_semaphore()

        @pl.when(my == 0)
        def _():
            for p in (1, 2, 3):
                pl.semaphore_signal(barrier, inc=1, device_id=(p,),
                                    device_id_type=_MESH)
            pl.semaphore_wait(barrier, 3)

            ck = pltpu.make_async_copy(
                k_hbm.at[0, :, pl.ds(0, HQ), :], kout, local_sems.at[0])
            cv = pltpu.make_async_copy(
                v_hbm.at[0, :, pl.ds(0, HQ), :], vout, local_sems.at[1])
            ck.start()
            cv.start()

            rdmas = []
            for p in (1, 2, 3):
                rk = pltpu.make_async_remote_copy(
                    src_ref=k_hbm.at[0, :, pl.ds(HQ * p, HQ), :],
                    dst_ref=kout,
                    send_sem=send_sems.at[2 * (p - 1)],
                    recv_sem=recv_sems.at[0],
                    device_id=(p,), device_id_type=_MESH)
                rv = pltpu.make_async_remote_copy(
                    src_ref=v_hbm.at[0, :, pl.ds(HQ * p, HQ), :],
                    dst_ref=vout,
                    send_sem=send_sems.at[2 * (p - 1) + 1],
                    recv_sem=recv_sems.at[1],
                    device_id=(p,), device_id_type=_MESH)
                rk.start()
                rv.start()
                rdmas.append(rk)
                rdmas.append(rv)

            ck.wait()
            cv.wait()
            for r in rdmas:
                r.wait_send()
            pl.semaphore_wait(credit_sem, 3)
            for p in (1, 2, 3):
                pl.semaphore_signal(ack_sem, inc=1, device_id=(p,),
                                    device_id_type=_MESH)

        @pl.when(my != 0)
        def _():
            pl.semaphore_signal(barrier, inc=1, device_id=(0,),
                                device_id_type=_MESH)
            pl.semaphore_wait(barrier, 1)
            rk = pltpu.make_async_remote_copy(
                src_ref=kout, dst_ref=kout,
                send_sem=send_sems.at[0], recv_sem=recv_sems.at[0],
                device_id=(0,), device_id_type=_MESH)
            rv = pltpu.make_async_remote_copy(
                src_ref=vout, dst_ref=vout,
                send_sem=send_sems.at[1], recv_sem=recv_sems.at[1],
                device_id=(0,), device_id_type=_MESH)
            rk.wait_recv()
            rv.wait_recv()
            pl.semaphore_signal(credit_sem, inc=1, device_id=(0,),
                                device_id_type=_MESH)
            pl.semaphore_wait(ack_sem, 1)

    return pl.pallas_call(
        body,
        out_shape=[
            jax.ShapeDtypeStruct((SKV, HQ, DH), jnp.float32),
            jax.ShapeDtypeStruct((SKV, HQ, DH), jnp.float32),
        ],
        in_specs=[
            pl.BlockSpec(memory_space=pl.ANY),
            pl.BlockSpec(memory_space=pl.ANY),
        ],
        out_specs=[
            pl.BlockSpec(memory_space=pltpu.VMEM),
            pl.BlockSpec(memory_space=pltpu.VMEM),
        ],
        scratch_shapes=[
            pltpu.SemaphoreType.DMA((2,)),
            pltpu.SemaphoreType.DMA((6,)),
            pltpu.SemaphoreType.DMA((2,)),
            pltpu.SemaphoreType.REGULAR,
            pltpu.SemaphoreType.REGULAR,
        ],
        compiler_params=pltpu.CompilerParams(collective_id=0),
    )(k_ext, v_ext)


def _qproj(x, wq):
    def body(x_ref, wq_ref, q_ref):
        q_ref[:, :] = jnp.dot(x_ref[0], wq_ref[:, :],
                              preferred_element_type=jnp.float32)

    return pl.pallas_call(
        body,
        out_shape=jax.ShapeDtypeStruct((SQ, DM), jnp.float32),
        in_specs=[
            pl.BlockSpec(memory_space=pltpu.VMEM),
            pl.BlockSpec(memory_space=pltpu.VMEM),
        ],
        out_specs=pl.BlockSpec(memory_space=pltpu.VMEM),
    )(x, wq)


def _attn(q, kh, vh):

    def body(q_ref, k_ref, v_ref, o_ref):
        qi = pl.program_id(1)
        s = lax.dot_general(
            q_ref[:, :], k_ref[:, :],
            (((1,), (1,)), ((), ())),
            preferred_element_type=jnp.float32) * SCALE
        r = qi * QBLK + lax.broadcasted_iota(jnp.int32, (QBLK, SKV), 0)
        c = lax.broadcasted_iota(jnp.int32, (QBLK, SKV), 1)
        s = jnp.where((c // MASK_BLK) <= (r // MASK_BLK), s, -1e9)
        m = jnp.max(s, axis=1, keepdims=True)
        w = jnp.exp(s - m)
        w = w / jnp.sum(w, axis=1, keepdims=True)
        o_ref[:, :] = jnp.dot(w, v_ref[:, :],
                              preferred_element_type=jnp.float32)

    return pl.pallas_call(
        body,
        grid=(HQ, SQ // QBLK),
        in_specs=[
            pl.BlockSpec((QBLK, DH), lambda h, qi: (qi, h)),
            pl.BlockSpec((SKV, DH), lambda h, qi: (0, h)),
            pl.BlockSpec((SKV, DH), lambda h, qi: (0, h)),
        ],
        out_specs=pl.BlockSpec((QBLK, DH), lambda h, qi: (qi, h)),
        out_shape=jax.ShapeDtypeStruct((SQ, DM), jnp.float32),
    )(q, kh, vh)


def _wo_allreduce(ctx, wo):

    def body(ctx_ref, wo_ref, out_ref, comm, send_sems, recv_sems):
        my = lax.axis_index("i")
        left = (my + N_DEV - 1) % N_DEV
        right = (my + 1) % N_DEV

        barrier = pltpu.get_barrier_semaphore()
        for nbr in (left, right):
            pl.semaphore_signal(barrier, inc=1, device_id=(nbr,),
                                device_id_type=_MESH)
        pl.semaphore_wait(barrier, 2)

        partial = jnp.dot(ctx_ref[:, :], wo_ref[:, :],
                          preferred_element_type=jnp.float32)
        out_ref[0] = partial
        comm[0] = partial

        for h in range(N_DEV - 1):
            send_slot = h % 2
            recv_slot = (h + 1) % 2
            rdma = pltpu.make_async_remote_copy(
                src_ref=comm.at[send_slot],
                dst_ref=comm.at[recv_slot],
                send_sem=send_sems.at[send_slot],
                recv_sem=recv_sems.at[recv_slot],
                device_id=(right,), device_id_type=_MESH)
            rdma.start()
            rdma.wait()
            out_ref[0] += comm[recv_slot]

    return pl.pallas_call(
        body,
        out_shape=jax.ShapeDtypeStruct((1, SQ, DM), jnp.float32),
        in_specs=[
            pl.BlockSpec(memory_space=pltpu.VMEM),
            pl.BlockSpec(memory_space=pltpu.VMEM),
        ],
        out_specs=pl.BlockSpec(memory_space=pltpu.VMEM),
        scratch_shapes=[
            pltpu.VMEM((2, SQ, DM), jnp.float32),
            pltpu.SemaphoreType.DMA((2,)),
            pltpu.SemaphoreType.DMA((2,)),
        ],
        compiler_params=pltpu.CompilerParams(collective_id=1),
    )(ctx, wo)


def kernel(x, Wq, K_ext, V_ext, Wo):
    kh, vh = _scatter_kv(K_ext, V_ext)
    q = _qproj(x, Wq)
    ctx = _attn(q, kh.reshape(SKV, HQ * DH), vh.reshape(SKV, HQ * DH))
    return _wo_allreduce(ctx, Wo)


# device time: 685025 ns/iter; 1.1345x vs baseline; 1.1345x over previous
import jax
import jax.numpy as jnp
from jax import lax
from jax.experimental import pallas as pl
from jax.experimental.pallas import tpu as pltpu

N_DEV = 4
SQ = 2048
SKV = 2048
HQ = 8
DH = 128
DM = 1024
SCALE = 0.08838834764831843
QBLK = 512
MASK_BLK = 64

_MESH = pl.DeviceIdType.MESH


KCHUNK = 512
NCHUNK = SKV // KCHUNK


def _fused_attn(x, wq, k_ext, v_ext):

    def body(x_ref, wq_ref, k_hbm, v_hbm, o_ref, kbuf, vbuf,
             local_sems, send_sems, recv_sems, credit_sem, ack_sem):
        my = lax.axis_index("i")
        barrier = pltpu.get_barrier_semaphore()

        @pl.when(my == 0)
        def _():
            for p in (1, 2, 3):
                pl.semaphore_signal(barrier, inc=1, device_id=(p,),
                                    device_id_type=_MESH)
            pl.semaphore_wait(barrier, 3)
            ck = pltpu.make_async_copy(
                k_hbm.at[0, :, pl.ds(0, HQ), :], kbuf, local_sems.at[0])
            cv = pltpu.make_async_copy(
                v_hbm.at[0, :, pl.ds(0, HQ), :], vbuf, local_sems.at[1])
            ck.start()
            cv.start()
            for c in range(NCHUNK):
                for p in (1, 2, 3):
                    i = (c * 3 + (p - 1)) * 2
                    rk = pltpu.make_async_remote_copy(
                        src_ref=k_hbm.at[0, pl.ds(KCHUNK * c, KCHUNK),
                                         pl.ds(HQ * p, HQ), :],
                        dst_ref=kbuf.at[pl.ds(KCHUNK * c, KCHUNK)],
                        send_sem=send_sems.at[i],
                        recv_sem=recv_sems.at[c],
                        device_id=(p,), device_id_type=_MESH)
                    rv = pltpu.make_async_remote_copy(
                        src_ref=v_hbm.at[0, pl.ds(KCHUNK * c, KCHUNK),
                                         pl.ds(HQ * p, HQ), :],
                        dst_ref=vbuf.at[pl.ds(KCHUNK * c, KCHUNK)],
                        send_sem=send_sems.at[i + 1],
                        recv_sem=recv_sems.at[NCHUNK + c],
                        device_id=(p,), device_id_type=_MESH)
                    rk.start()
                    rv.start()
            ck.wait()
            cv.wait()

        @pl.when(my != 0)
        def _():
            pl.semaphore_signal(barrier, inc=1, device_id=(0,),
                                device_id_type=_MESH)
            pl.semaphore_wait(barrier, 1)

        q_all = jnp.dot(x_ref[0], wq_ref[:, :],
                        preferred_element_type=jnp.float32)

        for c in range(NCHUNK):
            @pl.when(my != 0)
            def _(c=c):
                rk = pltpu.make_async_remote_copy(
                    src_ref=kbuf.at[pl.ds(KCHUNK * c, KCHUNK)],
                    dst_ref=kbuf.at[pl.ds(KCHUNK * c, KCHUNK)],
                    send_sem=send_sems.at[0], recv_sem=recv_sems.at[c],
                    device_id=(0,), device_id_type=_MESH)
                rv = pltpu.make_async_remote_copy(
                    src_ref=vbuf.at[pl.ds(KCHUNK * c, KCHUNK)],
                    dst_ref=vbuf.at[pl.ds(KCHUNK * c, KCHUNK)],
                    send_sem=send_sems.at[1], recv_sem=recv_sems.at[NCHUNK + c],
                    device_id=(0,), device_id_type=_MESH)
                rk.wait_recv()
                rv.wait_recv()

            L = KCHUNK * (c + 1)
            for h in range(HQ):
                q_h = q_all[KCHUNK * c:KCHUNK * (c + 1), DH * h:DH * (h + 1)]
                k_h = kbuf[pl.ds(0, L), h, :]
                v_h = vbuf[pl.ds(0, L), h, :]
                s = lax.dot_general(
                    q_h, k_h, (((1,), (1,)), ((), ())),
                    preferred_element_type=jnp.float32) * SCALE
                r = (KCHUNK * c
                     + lax.broadcasted_iota(jnp.int32, (KCHUNK, L), 0))
                cix = lax.broadcasted_iota(jnp.int32, (KCHUNK, L), 1)
                s = jnp.where((cix // MASK_BLK) <= (r // MASK_BLK), s, -1e9)
                m = jnp.max(s, axis=1, keepdims=True)
                w = jnp.exp(s - m)
                w = w / jnp.sum(w, axis=1, keepdims=True)
                o_ref[pl.ds(KCHUNK * c, KCHUNK), pl.ds(DH * h, DH)] = jnp.dot(
                    w, v_h, preferred_element_type=jnp.float32)

        @pl.when(my == 0)
        def _():
            for i in range(NCHUNK * 3 * 2):
                r = pltpu.make_async_remote_copy(
                    src_ref=kbuf.at[pl.ds(0, KCHUNK)],
                    dst_ref=kbuf.at[pl.ds(0, KCHUNK)],
                    send_sem=send_sems.at[i], recv_sem=recv_sems.at[0],
                    device_id=(1,), device_id_type=_MESH)
                r.wait_send()
            pl.semaphore_wait(credit_sem, 3)
            for p in (1, 2, 3):
                pl.semaphore_signal(ack_sem, inc=1, device_id=(p,),
                                    device_id_type=_MESH)

        @pl.when(my != 0)
        def _():
            pl.semaphore_signal(credit_sem, inc=1, device_id=(0,),
                                device_id_type=_MESH)
            pl.semaphore_wait(ack_sem, 1)

    return pl.pallas_call(
        body,
        out_shape=jax.ShapeDtypeStruct((SQ, DM), jnp.float32),
        in_specs=[
            pl.BlockSpec(memory_space=pltpu.VMEM),
            pl.BlockSpec(memory_space=pltpu.VMEM),
            pl.BlockSpec(memory_space=pl.ANY),
            pl.BlockSpec(memory_space=pl.ANY),
        ],
        out_specs=pl.BlockSpec(memory_space=pltpu.VMEM),
        scratch_shapes=[
            pltpu.VMEM((SKV, HQ, DH), jnp.float32),
            pltpu.VMEM((SKV, HQ, DH), jnp.float32),
            pltpu.SemaphoreType.DMA((2,)),
            pltpu.SemaphoreType.DMA((NCHUNK * 3 * 2,)),
            pltpu.SemaphoreType.DMA((2 * NCHUNK,)),
            pltpu.SemaphoreType.REGULAR,
            pltpu.SemaphoreType.REGULAR,
        ],
        compiler_params=pltpu.CompilerParams(collective_id=0),
    )(x, wq, k_ext, v_ext)


def _scatter_kv(k_ext, v_ext):

    def body(k_hbm, v_hbm, kout, vout, local_sems, send_sems, recv_sems,
             credit_sem, ack_sem):
        my = lax.axis_index("i")
        barrier = pltpu.get_barrier_semaphore()

        @pl.when(my == 0)
        def _():
            for p in (1, 2, 3):
                pl.semaphore_signal(barrier, inc=1, device_id=(p,),
                                    device_id_type=_MESH)
            pl.semaphore_wait(barrier, 3)

            ck = pltpu.make_async_copy(
                k_hbm.at[0, :, pl.ds(0, HQ), :], kout, local_sems.at[0])
            cv = pltpu.make_async_copy(
                v_hbm.at[0, :, pl.ds(0, HQ), :], vout, local_sems.at[1])
            ck.start()
            cv.start()

            rdmas = []
            for p in (1, 2, 3):
                rk = pltpu.make_async_remote_copy(
                    src_ref=k_hbm.at[0, :, pl.ds(HQ * p, HQ), :],
                    dst_ref=kout,
                    send_sem=send_sems.at[2 * (p - 1)],
                    recv_sem=recv_sems.at[0],
                    device_id=(p,), device_id_type=_MESH)
                rv = pltpu.make_async_remote_copy(
                    src_ref=v_hbm.at[0, :, pl.ds(HQ * p, HQ), :],
                    dst_ref=vout,
                    send_sem=send_sems.at[2 * (p - 1) + 1],
                    recv_sem=recv_sems.at[1],
                    device_id=(p,), device_id_type=_MESH)
                rk.start()
                rv.start()
                rdmas.append(rk)
                rdmas.append(rv)

            ck.wait()
            cv.wait()
            for r in rdmas:
                r.wait_send()
            pl.semaphore_wait(credit_sem, 3)
            for p in (1, 2, 3):
                pl.semaphore_signal(ack_sem, inc=1, device_id=(p,),
                                    device_id_type=_MESH)

        @pl.when(my != 0)
        def _():
            pl.semaphore_signal(barrier, inc=1, device_id=(0,),
                                device_id_type=_MESH)
            pl.semaphore_wait(barrier, 1)
            rk = pltpu.make_async_remote_copy(
                src_ref=kout, dst_ref=kout,
                send_sem=send_sems.at[0], recv_sem=recv_sems.at[0],
                device_id=(0,), device_id_type=_MESH)
            rv = pltpu.make_async_remote_copy(
                src_ref=vout, dst_ref=vout,
                send_sem=send_sems.at[1], recv_sem=recv_sems.at[1],
                device_id=(0,), device_id_type=_MESH)
            rk.wait_recv()
            rv.wait_recv()
            pl.semaphore_signal(credit_sem, inc=1, device_id=(0,),
                                device_id_type=_MESH)
            pl.semaphore_wait(ack_sem, 1)

    return pl.pallas_call(
        body,
        out_shape=[
            jax.ShapeDtypeStruct((SKV, HQ, DH), jnp.float32),
            jax.ShapeDtypeStruct((SKV, HQ, DH), jnp.float32),
        ],
        in_specs=[
            pl.BlockSpec(memory_space=pl.ANY),
            pl.BlockSpec(memory_space=pl.ANY),
        ],
        out_specs=[
            pl.BlockSpec(memory_space=pltpu.VMEM),
            pl.BlockSpec(memory_space=pltpu.VMEM),
        ],
        scratch_shapes=[
            pltpu.SemaphoreType.DMA((2,)),
            pltpu.SemaphoreType.DMA((6,)),
            pltpu.SemaphoreType.DMA((2,)),
            pltpu.SemaphoreType.REGULAR,
            pltpu.SemaphoreType.REGULAR,
        ],
        compiler_params=pltpu.CompilerParams(collective_id=0),
    )(k_ext, v_ext)


def _qproj(x, wq):
    def body(x_ref, wq_ref, q_ref):
        q_ref[:, :] = jnp.dot(x_ref[0], wq_ref[:, :],
                              preferred_element_type=jnp.float32)

    return pl.pallas_call(
        body,
        out_shape=jax.ShapeDtypeStruct((SQ, DM), jnp.float32),
        in_specs=[
            pl.BlockSpec(memory_space=pltpu.VMEM),
            pl.BlockSpec(memory_space=pltpu.VMEM),
        ],
        out_specs=pl.BlockSpec(memory_space=pltpu.VMEM),
    )(x, wq)


def _attn(q, kh, vh):

    def body(q_ref, k_ref, v_ref, o_ref):
        qi = pl.program_id(1)
        s = lax.dot_general(
            q_ref[:, :], k_ref[:, :],
            (((1,), (1,)), ((), ())),
            preferred_element_type=jnp.float32) * SCALE
        r = qi * QBLK + lax.broadcasted_iota(jnp.int32, (QBLK, SKV), 0)
        c = lax.broadcasted_iota(jnp.int32, (QBLK, SKV), 1)
        s = jnp.where((c // MASK_BLK) <= (r // MASK_BLK), s, -1e9)
        m = jnp.max(s, axis=1, keepdims=True)
        w = jnp.exp(s - m)
        w = w / jnp.sum(w, axis=1, keepdims=True)
        o_ref[:, :] = jnp.dot(w, v_ref[:, :],
                              preferred_element_type=jnp.float32)

    return pl.pallas_call(
        body,
        grid=(HQ, SQ // QBLK),
        in_specs=[
            pl.BlockSpec((QBLK, DH), lambda h, qi: (qi, h)),
            pl.BlockSpec((SKV, DH), lambda h, qi: (0, h)),
            pl.BlockSpec((SKV, DH), lambda h, qi: (0, h)),
        ],
        out_specs=pl.BlockSpec((QBLK, DH), lambda h, qi: (qi, h)),
        out_shape=jax.ShapeDtypeStruct((SQ, DM), jnp.float32),
    )(q, kh, vh)


def _wo_allreduce(ctx, wo):

    def body(ctx_ref, wo_ref, out_ref, comm, send_sems, recv_sems):
        my = lax.axis_index("i")
        left = (my + N_DEV - 1) % N_DEV
        right = (my + 1) % N_DEV

        barrier = pltpu.get_barrier_semaphore()
        for nbr in (left, right):
            pl.semaphore_signal(barrier, inc=1, device_id=(nbr,),
                                device_id_type=_MESH)
        pl.semaphore_wait(barrier, 2)

        partial = jnp.dot(ctx_ref[:, :], wo_ref[:, :],
                          preferred_element_type=jnp.float32)
        out_ref[0] = partial
        comm[0] = partial

        for h in range(N_DEV - 1):
            send_slot = h % 2
            recv_slot = (h + 1) % 2
            rdma = pltpu.make_async_remote_copy(
                src_ref=comm.at[send_slot],
                dst_ref=comm.at[recv_slot],
                send_sem=send_sems.at[send_slot],
                recv_sem=recv_sems.at[recv_slot],
                device_id=(right,), device_id_type=_MESH)
            rdma.start()
            rdma.wait()
            out_ref[0] += comm[recv_slot]

    return pl.pallas_call(
        body,
        out_shape=jax.ShapeDtypeStruct((1, SQ, DM), jnp.float32),
        in_specs=[
            pl.BlockSpec(memory_space=pltpu.VMEM),
            pl.BlockSpec(memory_space=pltpu.VMEM),
        ],
        out_specs=pl.BlockSpec(memory_space=pltpu.VMEM),
        scratch_shapes=[
            pltpu.VMEM((2, SQ, DM), jnp.float32),
            pltpu.SemaphoreType.DMA((2,)),
            pltpu.SemaphoreType.DMA((2,)),
        ],
        compiler_params=pltpu.CompilerParams(collective_id=1),
    )(ctx, wo)


def kernel(x, Wq, K_ext, V_ext, Wo):
    ctx = _fused_attn(x, Wq, K_ext, V_ext)
    return _wo_allreduce(ctx, Wo)


# device time: 550017 ns/iter; 1.4129x vs baseline; 1.2455x over previous
import jax
import jax.numpy as jnp
from jax import lax
from jax.experimental import pallas as pl
from jax.experimental.pallas import tpu as pltpu

N_DEV = 4
SQ = 2048
SKV = 2048
HQ = 8
DH = 128
DM = 1024
SCALE = 0.08838834764831843
QBLK = 512
MASK_BLK = 64

_MESH = pl.DeviceIdType.MESH


KCHUNK = 512
NCHUNK = SKV // KCHUNK


def _fused_attn(x, wq, k_ext, v_ext):

    def body(x_ref, wq_ref, k_hbm, v_hbm, o_ref, kbuf, vbuf,
             local_sems, send_sems, recv_sems, credit_sem, ack_sem):
        my = lax.axis_index("i")
        barrier = pltpu.get_barrier_semaphore()

        @pl.when(my == 0)
        def _():
            for p in (1, 2, 3):
                pl.semaphore_signal(barrier, inc=1, device_id=(p,),
                                    device_id_type=_MESH)
            pl.semaphore_wait(barrier, 3)
            ck = pltpu.make_async_copy(
                k_hbm.at[0, :, pl.ds(0, HQ), :], kbuf, local_sems.at[0])
            cv = pltpu.make_async_copy(
                v_hbm.at[0, :, pl.ds(0, HQ), :], vbuf, local_sems.at[1])
            ck.start()
            cv.start()
            for c in range(NCHUNK):
                for p in (1, 2, 3):
                    i = (c * 3 + (p - 1)) * 2
                    rk = pltpu.make_async_remote_copy(
                        src_ref=k_hbm.at[0, pl.ds(KCHUNK * c, KCHUNK),
                                         pl.ds(HQ * p, HQ), :],
                        dst_ref=kbuf.at[pl.ds(KCHUNK * c, KCHUNK)],
                        send_sem=send_sems.at[i],
                        recv_sem=recv_sems.at[c],
                        device_id=(p,), device_id_type=_MESH)
                    rv = pltpu.make_async_remote_copy(
                        src_ref=v_hbm.at[0, pl.ds(KCHUNK * c, KCHUNK),
                                         pl.ds(HQ * p, HQ), :],
                        dst_ref=vbuf.at[pl.ds(KCHUNK * c, KCHUNK)],
                        send_sem=send_sems.at[i + 1],
                        recv_sem=recv_sems.at[NCHUNK + c],
                        device_id=(p,), device_id_type=_MESH)
                    rk.start()
                    rv.start()
            ck.wait()
            cv.wait()

        @pl.when(my != 0)
        def _():
            pl.semaphore_signal(barrier, inc=1, device_id=(0,),
                                device_id_type=_MESH)
            pl.semaphore_wait(barrier, 1)

        q_all = jnp.dot(x_ref[0], wq_ref[:, :],
                        preferred_element_type=jnp.float32)

        for c in range(NCHUNK):
            @pl.when(my != 0)
            def _(c=c):
                rk = pltpu.make_async_remote_copy(
                    src_ref=kbuf.at[pl.ds(KCHUNK * c, KCHUNK)],
                    dst_ref=kbuf.at[pl.ds(KCHUNK * c, KCHUNK)],
                    send_sem=send_sems.at[0], recv_sem=recv_sems.at[c],
                    device_id=(0,), device_id_type=_MESH)
                rv = pltpu.make_async_remote_copy(
                    src_ref=vbuf.at[pl.ds(KCHUNK * c, KCHUNK)],
                    dst_ref=vbuf.at[pl.ds(KCHUNK * c, KCHUNK)],
                    send_sem=send_sems.at[1], recv_sem=recv_sems.at[NCHUNK + c],
                    device_id=(0,), device_id_type=_MESH)
                rk.wait_recv()
                rv.wait_recv()

            L = KCHUNK * (c + 1)
            for h in range(HQ):
                q_h = q_all[KCHUNK * c:KCHUNK * (c + 1), DH * h:DH * (h + 1)]
                k_h = kbuf[pl.ds(0, L), h, :]
                v_h = vbuf[pl.ds(0, L), h, :]
                s = lax.dot_general(
                    q_h, k_h, (((1,), (1,)), ((), ())),
                    preferred_element_type=jnp.float32) * SCALE
                r = (KCHUNK * c
                     + lax.broadcasted_iota(jnp.int32, (KCHUNK, L), 0))
                cix = lax.broadcasted_iota(jnp.int32, (KCHUNK, L), 1)
                s = jnp.where((cix // MASK_BLK) <= (r // MASK_BLK), s, -1e9)
                m = jnp.max(s, axis=1, keepdims=True)
                w = jnp.exp(s - m)
                w = w / jnp.sum(w, axis=1, keepdims=True)
                o_ref[pl.ds(KCHUNK * c, KCHUNK), pl.ds(DH * h, DH)] = jnp.dot(
                    w, v_h, preferred_element_type=jnp.float32)

        @pl.when(my == 0)
        def _():
            for i in range(NCHUNK * 3 * 2):
                r = pltpu.make_async_remote_copy(
                    src_ref=kbuf.at[pl.ds(0, KCHUNK)],
                    dst_ref=kbuf.at[pl.ds(0, KCHUNK)],
                    send_sem=send_sems.at[i], recv_sem=recv_sems.at[0],
                    device_id=(1,), device_id_type=_MESH)
                r.wait_send()
            pl.semaphore_wait(credit_sem, 3)
            for p in (1, 2, 3):
                pl.semaphore_signal(ack_sem, inc=1, device_id=(p,),
                                    device_id_type=_MESH)

        @pl.when(my != 0)
        def _():
            pl.semaphore_signal(credit_sem, inc=1, device_id=(0,),
                                device_id_type=_MESH)
            pl.semaphore_wait(ack_sem, 1)

    return pl.pallas_call(
        body,
        out_shape=jax.ShapeDtypeStruct((SQ, DM), jnp.float32),
        in_specs=[
            pl.BlockSpec(memory_space=pltpu.VMEM),
            pl.BlockSpec(memory_space=pltpu.VMEM),
            pl.BlockSpec(memory_space=pl.ANY),
            pl.BlockSpec(memory_space=pl.ANY),
        ],
        out_specs=pl.BlockSpec(memory_space=pltpu.VMEM),
        scratch_shapes=[
            pltpu.VMEM((SKV, HQ, DH), jnp.float32),
            pltpu.VMEM((SKV, HQ, DH), jnp.float32),
            pltpu.SemaphoreType.DMA((2,)),
            pltpu.SemaphoreType.DMA((NCHUNK * 3 * 2,)),
            pltpu.SemaphoreType.DMA((2 * NCHUNK,)),
            pltpu.SemaphoreType.REGULAR,
            pltpu.SemaphoreType.REGULAR,
        ],
        compiler_params=pltpu.CompilerParams(collective_id=0),
    )(x, wq, k_ext, v_ext)


def _scatter_kv(k_ext, v_ext):

    def body(k_hbm, v_hbm, kout, vout, local_sems, send_sems, recv_sems,
             credit_sem, ack_sem):
        my = lax.axis_index("i")
        barrier = pltpu.get_barrier_semaphore()

        @pl.when(my == 0)
        def _():
            for p in (1, 2, 3):
                pl.semaphore_signal(barrier, inc=1, device_id=(p,),
                                    device_id_type=_MESH)
            pl.semaphore_wait(barrier, 3)

            ck = pltpu.make_async_copy(
                k_hbm.at[0, :, pl.ds(0, HQ), :], kout, local_sems.at[0])
            cv = pltpu.make_async_copy(
                v_hbm.at[0, :, pl.ds(0, HQ), :], vout, local_sems.at[1])
            ck.start()
            cv.start()

            rdmas = []
            for p in (1, 2, 3):
                rk = pltpu.make_async_remote_copy(
                    src_ref=k_hbm.at[0, :, pl.ds(HQ * p, HQ), :],
                    dst_ref=kout,
                    send_sem=send_sems.at[2 * (p - 1)],
                    recv_sem=recv_sems.at[0],
                    device_id=(p,), device_id_type=_MESH)
                rv = pltpu.make_async_remote_copy(
                    src_ref=v_hbm.at[0, :, pl.ds(HQ * p, HQ), :],
                    dst_ref=vout,
                    send_sem=send_sems.at[2 * (p - 1) + 1],
                    recv_sem=recv_sems.at[1],
                    device_id=(p,), device_id_type=_MESH)
                rk.start()
                rv.start()
                rdmas.append(rk)
                rdmas.append(rv)

            ck.wait()
            cv.wait()
            for r in rdmas:
                r.wait_send()
            pl.semaphore_wait(credit_sem, 3)
            for p in (1, 2, 3):
                pl.semaphore_signal(ack_sem, inc=1, device_id=(p,),
                                    device_id_type=_MESH)

        @pl.when(my != 0)
        def _():
            pl.semaphore_signal(barrier, inc=1, device_id=(0,),
                                device_id_type=_MESH)
            pl.semaphore_wait(barrier, 1)
            rk = pltpu.make_async_remote_copy(
                src_ref=kout, dst_ref=kout,
                send_sem=send_sems.at[0], recv_sem=recv_sems.at[0],
                device_id=(0,), device_id_type=_MESH)
            rv = pltpu.make_async_remote_copy(
                src_ref=vout, dst_ref=vout,
                send_sem=send_sems.at[1], recv_sem=recv_sems.at[1],
                device_id=(0,), device_id_type=_MESH)
            rk.wait_recv()
            rv.wait_recv()
            pl.semaphore_signal(credit_sem, inc=1, device_id=(0,),
                                device_id_type=_MESH)
            pl.semaphore_wait(ack_sem, 1)

    return pl.pallas_call(
        body,
        out_shape=[
            jax.ShapeDtypeStruct((SKV, HQ, DH), jnp.float32),
            jax.ShapeDtypeStruct((SKV, HQ, DH), jnp.float32),
        ],
        in_specs=[
            pl.BlockSpec(memory_space=pl.ANY),
            pl.BlockSpec(memory_space=pl.ANY),
        ],
        out_specs=[
            pl.BlockSpec(memory_space=pltpu.VMEM),
            pl.BlockSpec(memory_space=pltpu.VMEM),
        ],
        scratch_shapes=[
            pltpu.SemaphoreType.DMA((2,)),
            pltpu.SemaphoreType.DMA((6,)),
            pltpu.SemaphoreType.DMA((2,)),
            pltpu.SemaphoreType.REGULAR,
            pltpu.SemaphoreType.REGULAR,
        ],
        compiler_params=pltpu.CompilerParams(collective_id=0),
    )(k_ext, v_ext)


def _qproj(x, wq):
    def body(x_ref, wq_ref, q_ref):
        q_ref[:, :] = jnp.dot(x_ref[0], wq_ref[:, :],
                              preferred_element_type=jnp.float32)

    return pl.pallas_call(
        body,
        out_shape=jax.ShapeDtypeStruct((SQ, DM), jnp.float32),
        in_specs=[
            pl.BlockSpec(memory_space=pltpu.VMEM),
            pl.BlockSpec(memory_space=pltpu.VMEM),
        ],
        out_specs=pl.BlockSpec(memory_space=pltpu.VMEM),
    )(x, wq)


def _attn(q, kh, vh):

    def body(q_ref, k_ref, v_ref, o_ref):
        qi = pl.program_id(1)
        s = lax.dot_general(
            q_ref[:, :], k_ref[:, :],
            (((1,), (1,)), ((), ())),
            preferred_element_type=jnp.float32) * SCALE
        r = qi * QBLK + lax.broadcasted_iota(jnp.int32, (QBLK, SKV), 0)
        c = lax.broadcasted_iota(jnp.int32, (QBLK, SKV), 1)
        s = jnp.where((c // MASK_BLK) <= (r // MASK_BLK), s, -1e9)
        m = jnp.max(s, axis=1, keepdims=True)
        w = jnp.exp(s - m)
        w = w / jnp.sum(w, axis=1, keepdims=True)
        o_ref[:, :] = jnp.dot(w, v_ref[:, :],
                              preferred_element_type=jnp.float32)

    return pl.pallas_call(
        body,
        grid=(HQ, SQ // QBLK),
        in_specs=[
            pl.BlockSpec((QBLK, DH), lambda h, qi: (qi, h)),
            pl.BlockSpec((SKV, DH), lambda h, qi: (0, h)),
            pl.BlockSpec((SKV, DH), lambda h, qi: (0, h)),
        ],
        out_specs=pl.BlockSpec((QBLK, DH), lambda h, qi: (qi, h)),
        out_shape=jax.ShapeDtypeStruct((SQ, DM), jnp.float32),
    )(q, kh, vh)


RCH = SQ // N_DEV


def _wo_allreduce(ctx, wo):

    def body(ctx_ref, wo_ref, out_ref, comm, send_sems, recv_sems):
        my = lax.axis_index("i")
        left = (my + N_DEV - 1) % N_DEV
        right = (my + 1) % N_DEV

        barrier = pltpu.get_barrier_semaphore()
        for nbr in (left, right):
            pl.semaphore_signal(barrier, inc=1, device_id=(nbr,),
                                device_id_type=_MESH)
        pl.semaphore_wait(barrier, 2)

        def pchunk(idx):
            off = (idx % N_DEV) * RCH
            return jnp.dot(ctx_ref[pl.ds(off, RCH), :], wo_ref[:, :],
                           preferred_element_type=jnp.float32)

        comm[0] = pchunk(my + N_DEV - 1)
        for s in range(N_DEV - 1):
            rdma = pltpu.make_async_remote_copy(
                src_ref=comm.at[s % 2],
                dst_ref=comm.at[(s + 1) % 2],
                send_sem=send_sems.at[s % 2],
                recv_sem=recv_sems.at[(s + 1) % 2],
                device_id=(right,), device_id_type=_MESH)
            rdma.start()
            pc = pchunk(my + 2 * N_DEV - 2 - s)
            rdma.wait()
            comm[(s + 1) % 2] = comm[(s + 1) % 2] + pc

        out_ref[0, pl.ds(my * RCH, RCH), :] = comm[1]

        for t in range(N_DEV - 1):
            src_slot = (1 + t) % 2
            dst_slot = (t % 2)
            rdma = pltpu.make_async_remote_copy(
                src_ref=comm.at[src_slot],
                dst_ref=comm.at[dst_slot],
                send_sem=send_sems.at[src_slot],
                recv_sem=recv_sems.at[dst_slot],
                device_id=(right,), device_id_type=_MESH)
            rdma.start()
            rdma.wait()
            idx = (my + N_DEV - 1 - t) % N_DEV
            out_ref[0, pl.ds(idx * RCH, RCH), :] = comm[dst_slot]

    return pl.pallas_call(
        body,
        out_shape=jax.ShapeDtypeStruct((1, SQ, DM), jnp.float32),
        in_specs=[
            pl.BlockSpec(memory_space=pltpu.VMEM),
            pl.BlockSpec(memory_space=pltpu.VMEM),
        ],
        out_specs=pl.BlockSpec(memory_space=pltpu.VMEM),
        scratch_shapes=[
            pltpu.VMEM((2, RCH, DM), jnp.float32),
            pltpu.SemaphoreType.DMA((2,)),
            pltpu.SemaphoreType.DMA((2,)),
        ],
        compiler_params=pltpu.CompilerParams(collective_id=1),
    )(ctx, wo)


def kernel(x, Wq, K_ext, V_ext, Wo):
    ctx = _fused_attn(x, Wq, K_ext, V_ext)
    return _wo_allreduce(ctx, Wo)


# device time: 465468 ns/iter; 1.6696x vs baseline; 1.1816x over previous
import jax
import jax.numpy as jnp
from jax import lax
from jax.experimental import pallas as pl
from jax.experimental.pallas import tpu as pltpu

N_DEV = 4
SQ = 2048
SKV = 2048
HQ = 8
DH = 128
DM = 1024
SCALE = 0.08838834764831843
MASK_BLK = 64
KCHUNK = 512
NCHUNK = SKV // KCHUNK
RCH = SQ // N_DEV

_MESH = pl.DeviceIdType.MESH


def _fused_attn(x, wq, k_ext, v_ext):

    def body(x_ref, wq_ref, k_hbm, v_hbm, o_ref, kbuf, vbuf, relay,
             local_sems, send_sems, recv_sems, relay_recv, fwd_sems,
             credit_sem, ack_sem, p2credit_sem):
        my = lax.axis_index("i")
        barrier = pltpu.get_barrier_semaphore()

        def kchunk(buf, c):
            return buf.at[pl.ds(KCHUNK * c, KCHUNK)]

        @pl.when(my == 0)
        def _():
            for p in (1, 2, 3):
                pl.semaphore_signal(barrier, inc=1, device_id=(p,),
                                    device_id_type=_MESH)
            pl.semaphore_wait(barrier, 3)
            ck = pltpu.make_async_copy(
                k_hbm.at[0, :, pl.ds(0, HQ), :], kbuf, local_sems.at[0])
            cv = pltpu.make_async_copy(
                v_hbm.at[0, :, pl.ds(0, HQ), :], vbuf, local_sems.at[1])
            ck.start()
            cv.start()
            for c in range(NCHUNK):
                b = c * 6
                sends = [
                    (v_hbm, 2 * HQ, relay, relay_recv.at[c], 1),
                    (k_hbm, 2 * HQ, relay, relay_recv.at[c], 3),
                    (k_hbm, 1 * HQ, kbuf, recv_sems.at[c], 1),
                    (v_hbm, 1 * HQ, vbuf, recv_sems.at[NCHUNK + c], 1),
                    (k_hbm, 3 * HQ, kbuf, recv_sems.at[c], 3),
                    (v_hbm, 3 * HQ, vbuf, recv_sems.at[NCHUNK + c], 3),
                ]
                for j, (src, h0, dst, rsem, peer) in enumerate(sends):
                    r = pltpu.make_async_remote_copy(
                        src_ref=src.at[0, pl.ds(KCHUNK * c, KCHUNK),
                                       pl.ds(h0, HQ), :],
                        dst_ref=kchunk(dst, c),
                        send_sem=send_sems.at[b + j],
                        recv_sem=rsem,
                        device_id=(peer,), device_id_type=_MESH)
                    r.start()
            ck.wait()
            cv.wait()

        @pl.when(my != 0)
        def _():
            pl.semaphore_signal(barrier, inc=1, device_id=(0,),
                                device_id_type=_MESH)
            pl.semaphore_wait(barrier, 1)

        q_all = jnp.dot(x_ref[0], wq_ref[:, :],
                        preferred_element_type=jnp.float32)

        for c in range(NCHUNK):
            @pl.when(my == 1)
            def _(c=c):
                pltpu.make_async_remote_copy(
                    src_ref=kchunk(relay, c), dst_ref=kchunk(relay, c),
                    send_sem=fwd_sems.at[c], recv_sem=relay_recv.at[c],
                    device_id=(0,), device_id_type=_MESH).wait_recv()
                pltpu.make_async_remote_copy(
                    src_ref=kchunk(relay, c), dst_ref=kchunk(vbuf, c),
                    send_sem=fwd_sems.at[c],
                    recv_sem=recv_sems.at[NCHUNK + c],
                    device_id=(2,), device_id_type=_MESH).start()

            @pl.when(my == 3)
            def _(c=c):
                pltpu.make_async_remote_copy(
                    src_ref=kchunk(relay, c), dst_ref=kchunk(relay, c),
                    send_sem=fwd_sems.at[c], recv_sem=relay_recv.at[c],
                    device_id=(0,), device_id_type=_MESH).wait_recv()
                pltpu.make_async_remote_copy(
                    src_ref=kchunk(relay, c), dst_ref=kchunk(kbuf, c),
                    send_sem=fwd_sems.at[c],
                    recv_sem=recv_sems.at[c],
                    device_id=(2,), device_id_type=_MESH).start()

            @pl.when(my != 0)
            def _(c=c):
                pltpu.make_async_remote_copy(
                    src_ref=kchunk(kbuf, c), dst_ref=kchunk(kbuf, c),
                    send_sem=send_sems.at[0], recv_sem=recv_sems.at[c],
                    device_id=(0,), device_id_type=_MESH).wait_recv()
                pltpu.make_async_remote_copy(
                    src_ref=kchunk(vbuf, c), dst_ref=kchunk(vbuf, c),
                    send_sem=send_sems.at[1],
                    recv_sem=recv_sems.at[NCHUNK + c],
                    device_id=(0,), device_id_type=_MESH).wait_recv()

            L = KCHUNK * (c + 1)
            for h in range(HQ):
                q_h = q_all[KCHUNK * c:KCHUNK * (c + 1), DH * h:DH * (h + 1)]
                k_h = kbuf[pl.ds(0, L), h, :]
                v_h = vbuf[pl.ds(0, L), h, :]
                s = lax.dot_general(
                    q_h, k_h, (((1,), (1,)), ((), ())),
                    preferred_element_type=jnp.float32) * SCALE
                r = (KCHUNK * c
                     + lax.broadcasted_iota(jnp.int32, (KCHUNK, L), 0))
                cix = lax.broadcasted_iota(jnp.int32, (KCHUNK, L), 1)
                s = jnp.where((cix // MASK_BLK) <= (r // MASK_BLK), s, -1e9)
                m = jnp.max(s, axis=1, keepdims=True)
                w = jnp.exp(s - m)
                w = w / jnp.sum(w, axis=1, keepdims=True)
                o_ref[pl.ds(KCHUNK * c, KCHUNK), pl.ds(DH * h, DH)] = jnp.dot(
                    w, v_h, preferred_element_type=jnp.float32)

        @pl.when(my == 0)
        def _():
            for i in range(6 * NCHUNK):
                pltpu.make_async_remote_copy(
                    src_ref=kchunk(kbuf, 0), dst_ref=kchunk(kbuf, 0),
                    send_sem=send_sems.at[i], recv_sem=recv_sems.at[0],
                    device_id=(1,), device_id_type=_MESH).wait_send()
            pl.semaphore_wait(credit_sem, 2)
            for p in (1, 3):
                pl.semaphore_signal(ack_sem, inc=1, device_id=(p,),
                                    device_id_type=_MESH)

        @pl.when(jnp.logical_or(my == 1, my == 3))
        def _():
            for c in range(NCHUNK):
                pltpu.make_async_remote_copy(
                    src_ref=kchunk(relay, c), dst_ref=kchunk(relay, c),
                    send_sem=fwd_sems.at[c], recv_sem=recv_sems.at[0],
                    device_id=(2,), device_id_type=_MESH).wait_send()
            pl.semaphore_signal(credit_sem, inc=1, device_id=(0,),
                                device_id_type=_MESH)
            pl.semaphore_wait(p2credit_sem, 1)
            pl.semaphore_wait(ack_sem, 1)

        @pl.when(my == 2)
        def _():
            for p in (1, 3):
                pl.semaphore_signal(p2credit_sem, inc=1, device_id=(p,),
                                    device_id_type=_MESH)

    return pl.pallas_call(
        body,
        out_shape=jax.ShapeDtypeStruct((SQ, DM), jnp.float32),
        in_specs=[
            pl.BlockSpec(memory_space=pltpu.VMEM),
            pl.BlockSpec(memory_space=pltpu.VMEM),
            pl.BlockSpec(memory_space=pl.ANY),
            pl.BlockSpec(memory_space=pl.ANY),
        ],
        out_specs=pl.BlockSpec(memory_space=pltpu.VMEM),
        scratch_shapes=[
            pltpu.VMEM((SKV, HQ, DH), jnp.float32),
            pltpu.VMEM((SKV, HQ, DH), jnp.float32),
            pltpu.VMEM((SKV, HQ, DH), jnp.float32),
            pltpu.SemaphoreType.DMA((2,)),
            pltpu.SemaphoreType.DMA((6 * NCHUNK,)),
            pltpu.SemaphoreType.DMA((2 * NCHUNK,)),
            pltpu.SemaphoreType.DMA((NCHUNK,)),
            pltpu.SemaphoreType.DMA((NCHUNK,)),
            pltpu.SemaphoreType.REGULAR,
            pltpu.SemaphoreType.REGULAR,
            pltpu.SemaphoreType.REGULAR,
        ],
        compiler_params=pltpu.CompilerParams(
            collective_id=0, vmem_limit_bytes=100 * 1024 * 1024),
    )(x, wq, k_ext, v_ext)


def _wo_allreduce(ctx, wo):

    def body(ctx_ref, wo_ref, out_ref, comm, send_sems, recv_sems):
        my = lax.axis_index("i")
        left = (my + N_DEV - 1) % N_DEV
        right = (my + 1) % N_DEV

        barrier = pltpu.get_barrier_semaphore()
        for nbr in (left, right):
            pl.semaphore_signal(barrier, inc=1, device_id=(nbr,),
                                device_id_type=_MESH)
        pl.semaphore_wait(barrier, 2)

        def pchunk(idx):
            off = (idx % N_DEV) * RCH
            return jnp.dot(ctx_ref[pl.ds(off, RCH), :], wo_ref[:, :],
                           preferred_element_type=jnp.float32)

        comm[0] = pchunk(my + N_DEV - 1)
        for s in range(N_DEV - 1):
            rdma = pltpu.make_async_remote_copy(
                src_ref=comm.at[s % 2],
                dst_ref=comm.at[(s + 1) % 2],
                send_sem=send_sems.at[s % 2],
                recv_sem=recv_sems.at[(s + 1) % 2],
                device_id=(right,), device_id_type=_MESH)
            rdma.start()
            pc = pchunk(my + 2 * N_DEV - 2 - s)
            rdma.wait()
            comm[(s + 1) % 2] = comm[(s + 1) % 2] + pc

        out_ref[0, pl.ds(my * RCH, RCH), :] = comm[1]

        for t in range(N_DEV - 1):
            src_slot = (1 + t) % 2
            dst_slot = (t % 2)
            rdma = pltpu.make_async_remote_copy(
                src_ref=comm.at[src_slot],
                dst_ref=comm.at[dst_slot],
                send_sem=send_sems.at[src_slot],
                recv_sem=recv_sems.at[dst_slot],
                device_id=(right,), device_id_type=_MESH)
            rdma.start()
            rdma.wait()
            idx = (my + N_DEV - 1 - t) % N_DEV
            out_ref[0, pl.ds(idx * RCH, RCH), :] = comm[dst_slot]

    return pl.pallas_call(
        body,
        out_shape=jax.ShapeDtypeStruct((1, SQ, DM), jnp.float32),
        in_specs=[
            pl.BlockSpec(memory_space=pltpu.VMEM),
            pl.BlockSpec(memory_space=pltpu.VMEM),
        ],
        out_specs=pl.BlockSpec(memory_space=pltpu.VMEM),
        scratch_shapes=[
            pltpu.VMEM((2, RCH, DM), jnp.float32),
            pltpu.SemaphoreType.DMA((2,)),
            pltpu.SemaphoreType.DMA((2,)),
        ],
        compiler_params=pltpu.CompilerParams(collective_id=1),
    )(ctx, wo)


def kernel(x, Wq, K_ext, V_ext, Wo):
    ctx = _fused_attn(x, Wq, K_ext, V_ext)
    return _wo_allreduce(ctx, Wo)


# device time: 378445 ns/iter; 2.0535x vs baseline; 1.2299x over previous
import jax
import jax.numpy as jnp
from jax import lax
from jax.experimental import pallas as pl
from jax.experimental.pallas import tpu as pltpu

N_DEV = 4
SQ = 2048
SKV = 2048
HQ = 8
DH = 128
DM = 1024
SCALE = 0.08838834764831843
MASK_BLK = 64
KCHUNK = 1024
NCHUNK = SKV // KCHUNK
RCH = SQ // N_DEV

_MESH = pl.DeviceIdType.MESH


def _to_bf16(k_ext, v_ext):

    def body(k_ref, v_ref, ko_ref, vo_ref):
        ko_ref[...] = k_ref[...].astype(jnp.bfloat16)
        vo_ref[...] = v_ref[...].astype(jnp.bfloat16)

    n = 16
    blk = pl.BlockSpec((1, SKV // n, 4 * HQ, DH), lambda i: (0, i, 0, 0))
    return pl.pallas_call(
        body,
        grid=(n,),
        in_specs=[blk, blk],
        out_specs=[blk, blk],
        out_shape=[
            jax.ShapeDtypeStruct((1, SKV, 4 * HQ, DH), jnp.bfloat16),
            jax.ShapeDtypeStruct((1, SKV, 4 * HQ, DH), jnp.bfloat16),
        ],
    )(k_ext, v_ext)


def _fused_attn(x, wq, k_ext, v_ext):

    def body(x_ref, wq_ref, k_hbm, v_hbm, o_ref, kbuf, vbuf, relay,
             local_sems, send_sems, recv_sems, relay_recv, fwd_sems,
             credit_sem, ack_sem, p2credit_sem):
        my = lax.axis_index("i")
        barrier = pltpu.get_barrier_semaphore()

        def kchunk(buf, c):
            return buf.at[pl.ds(KCHUNK * c, KCHUNK)]

        @pl.when(my == 0)
        def _():
            for p in (1, 2, 3):
                pl.semaphore_signal(barrier, inc=1, device_id=(p,),
                                    device_id_type=_MESH)
            pl.semaphore_wait(barrier, 3)
            ck = pltpu.make_async_copy(
                k_hbm.at[0, :, pl.ds(0, HQ), :], kbuf, local_sems.at[0])
            cv = pltpu.make_async_copy(
                v_hbm.at[0, :, pl.ds(0, HQ), :], vbuf, local_sems.at[1])
            ck.start()
            cv.start()
            for c in range(NCHUNK):
                b = c * 6
                sends = [
                    (v_hbm, 2 * HQ, relay, relay_recv.at[c], 1),
                    (k_hbm, 2 * HQ, relay, relay_recv.at[c], 3),
                    (k_hbm, 1 * HQ, kbuf, recv_sems.at[c], 1),
                    (v_hbm, 1 * HQ, vbuf, recv_sems.at[NCHUNK + c], 1),
                    (k_hbm, 3 * HQ, kbuf, recv_sems.at[c], 3),
                    (v_hbm, 3 * HQ, vbuf, recv_sems.at[NCHUNK + c], 3),
                ]
                for j, (src, h0, dst, rsem, peer) in enumerate(sends):
                    r = pltpu.make_async_remote_copy(
                        src_ref=src.at[0, pl.ds(KCHUNK * c, KCHUNK),
                                       pl.ds(h0, HQ), :],
                        dst_ref=kchunk(dst, c),
                        send_sem=send_sems.at[b + j],
                        recv_sem=rsem,
                        device_id=(peer,), device_id_type=_MESH)
                    r.start()
            ck.wait()
            cv.wait()

        @pl.when(my != 0)
        def _():
            pl.semaphore_signal(barrier, inc=1, device_id=(0,),
                                device_id_type=_MESH)
            pl.semaphore_wait(barrier, 1)

        q_all = jnp.dot(x_ref[0].astype(jnp.bfloat16),
                        wq_ref[:, :].astype(jnp.bfloat16),
                        preferred_element_type=jnp.float32)

        for c in range(NCHUNK):
            @pl.when(my == 1)
            def _(c=c):
                pltpu.make_async_remote_copy(
                    src_ref=kchunk(relay, c), dst_ref=kchunk(relay, c),
                    send_sem=fwd_sems.at[c], recv_sem=relay_recv.at[c],
                    device_id=(0,), device_id_type=_MESH).wait_recv()
                pltpu.make_async_remote_copy(
                    src_ref=kchunk(relay, c), dst_ref=kchunk(vbuf, c),
                    send_sem=fwd_sems.at[c],
                    recv_sem=recv_sems.at[NCHUNK + c],
                    device_id=(2,), device_id_type=_MESH).start()

            @pl.when(my == 3)
            def _(c=c):
                pltpu.make_async_remote_copy(
                    src_ref=kchunk(relay, c), dst_ref=kchunk(relay, c),
                    send_sem=fwd_sems.at[c], recv_sem=relay_recv.at[c],
                    device_id=(0,), device_id_type=_MESH).wait_recv()
                pltpu.make_async_remote_copy(
                    src_ref=kchunk(relay, c), dst_ref=kchunk(kbuf, c),
                    send_sem=fwd_sems.at[c],
                    recv_sem=recv_sems.at[c],
                    device_id=(2,), device_id_type=_MESH).start()

            @pl.when(my != 0)
            def _(c=c):
                pltpu.make_async_remote_copy(
                    src_ref=kchunk(kbuf, c), dst_ref=kchunk(kbuf, c),
                    send_sem=send_sems.at[0], recv_sem=recv_sems.at[c],
                    device_id=(0,), device_id_type=_MESH).wait_recv()
                pltpu.make_async_remote_copy(
                    src_ref=kchunk(vbuf, c), dst_ref=kchunk(vbuf, c),
                    send_sem=send_sems.at[1],
                    recv_sem=recv_sems.at[NCHUNK + c],
                    device_id=(0,), device_id_type=_MESH).wait_recv()

            L = KCHUNK * (c + 1)
            for h in range(HQ):
                q_h = q_all[KCHUNK * c:KCHUNK * (c + 1),
                            DH * h:DH * (h + 1)].astype(jnp.bfloat16)
                k_h = kbuf[pl.ds(0, L), h, :]
                v_h = vbuf[pl.ds(0, L), h, :]
                s = lax.dot_general(
                    q_h, k_h, (((1,), (1,)), ((), ())),
                    preferred_element_type=jnp.float32) * SCALE
                r = (KCHUNK * c
                     + lax.broadcasted_iota(jnp.int32, (KCHUNK, L), 0))
                cix = lax.broadcasted_iota(jnp.int32, (KCHUNK, L), 1)
                s = jnp.where((cix // MASK_BLK) <= (r // MASK_BLK), s, -1e9)
                m = jnp.max(s, axis=1, keepdims=True)
                w = jnp.exp(s - m)
                w = (w / jnp.sum(w, axis=1, keepdims=True)).astype(jnp.bfloat16)
                o_ref[pl.ds(KCHUNK * c, KCHUNK), pl.ds(DH * h, DH)] = jnp.dot(
                    w, v_h, preferred_element_type=jnp.float32)

        @pl.when(my == 0)
        def _():
            for i in range(6 * NCHUNK):
                pltpu.make_async_remote_copy(
                    src_ref=kchunk(kbuf, 0), dst_ref=kchunk(kbuf, 0),
                    send_sem=send_sems.at[i], recv_sem=recv_sems.at[0],
                    device_id=(1,), device_id_type=_MESH).wait_send()
            pl.semaphore_wait(credit_sem, 2)
            for p in (1, 3):
                pl.semaphore_signal(ack_sem, inc=1, device_id=(p,),
                                    device_id_type=_MESH)

        @pl.when(jnp.logical_or(my == 1, my == 3))
        def _():
            for c in range(NCHUNK):
                pltpu.make_async_remote_copy(
                    src_ref=kchunk(relay, c), dst_ref=kchunk(relay, c),
                    send_sem=fwd_sems.at[c], recv_sem=recv_sems.at[0],
                    device_id=(2,), device_id_type=_MESH).wait_send()
            pl.semaphore_signal(credit_sem, inc=1, device_id=(0,),
                                device_id_type=_MESH)
            pl.semaphore_wait(p2credit_sem, 1)
            pl.semaphore_wait(ack_sem, 1)

        @pl.when(my == 2)
        def _():
            for p in (1, 3):
                pl.semaphore_signal(p2credit_sem, inc=1, device_id=(p,),
                                    device_id_type=_MESH)

    return pl.pallas_call(
        body,
        out_shape=jax.ShapeDtypeStruct((SQ, DM), jnp.float32),
        in_specs=[
            pl.BlockSpec(memory_space=pltpu.VMEM),
            pl.BlockSpec(memory_space=pltpu.VMEM),
            pl.BlockSpec(memory_space=pl.ANY),
            pl.BlockSpec(memory_space=pl.ANY),
        ],
        out_specs=pl.BlockSpec(memory_space=pltpu.VMEM),
        scratch_shapes=[
            pltpu.VMEM((SKV, HQ, DH), jnp.bfloat16),
            pltpu.VMEM((SKV, HQ, DH), jnp.bfloat16),
            pltpu.VMEM((SKV, HQ, DH), jnp.bfloat16),
            pltpu.SemaphoreType.DMA((2,)),
            pltpu.SemaphoreType.DMA((6 * NCHUNK,)),
            pltpu.SemaphoreType.DMA((2 * NCHUNK,)),
            pltpu.SemaphoreType.DMA((NCHUNK,)),
            pltpu.SemaphoreType.DMA((NCHUNK,)),
            pltpu.SemaphoreType.REGULAR,
            pltpu.SemaphoreType.REGULAR,
            pltpu.SemaphoreType.REGULAR,
        ],
        compiler_params=pltpu.CompilerParams(
            collective_id=0, vmem_limit_bytes=100 * 1024 * 1024),
    )(x, wq, k_ext, v_ext)


def _wo_allreduce(ctx, wo):

    def body(ctx_ref, wo_ref, out_ref, comm, send_sems, recv_sems):
        my = lax.axis_index("i")
        left = (my + N_DEV - 1) % N_DEV
        right = (my + 1) % N_DEV

        barrier = pltpu.get_barrier_semaphore()
        for nbr in (left, right):
            pl.semaphore_signal(barrier, inc=1, device_id=(nbr,),
                                device_id_type=_MESH)
        pl.semaphore_wait(barrier, 2)

        def pchunk(idx):
            off = (idx % N_DEV) * RCH
            return jnp.dot(ctx_ref[pl.ds(off, RCH), :], wo_ref[:, :],
                           preferred_element_type=jnp.float32)

        comm[0] = pchunk(my + N_DEV - 1)
        for s in range(N_DEV - 1):
            rdma = pltpu.make_async_remote_copy(
                src_ref=comm.at[s % 2],
                dst_ref=comm.at[(s + 1) % 2],
                send_sem=send_sems.at[s % 2],
                recv_sem=recv_sems.at[(s + 1) % 2],
                device_id=(right,), device_id_type=_MESH)
            rdma.start()
            pc = pchunk(my + 2 * N_DEV - 2 - s)
            rdma.wait()
            comm[(s + 1) % 2] = comm[(s + 1) % 2] + pc

        out_ref[0, pl.ds(my * RCH, RCH), :] = comm[1]

        for t in range(N_DEV - 1):
            src_slot = (1 + t) % 2
            dst_slot = (t % 2)
            rdma = pltpu.make_async_remote_copy(
                src_ref=comm.at[src_slot],
                dst_ref=comm.at[dst_slot],
                send_sem=send_sems.at[src_slot],
                recv_sem=recv_sems.at[dst_slot],
                device_id=(right,), device_id_type=_MESH)
            rdma.start()
            rdma.wait()
            idx = (my + N_DEV - 1 - t) % N_DEV
            out_ref[0, pl.ds(idx * RCH, RCH), :] = comm[dst_slot]

    return pl.pallas_call(
        body,
        out_shape=jax.ShapeDtypeStruct((1, SQ, DM), jnp.float32),
        in_specs=[
            pl.BlockSpec(memory_space=pltpu.VMEM),
            pl.BlockSpec(memory_space=pltpu.VMEM),
        ],
        out_specs=pl.BlockSpec(memory_space=pltpu.VMEM),
        scratch_shapes=[
            pltpu.VMEM((2, RCH, DM), jnp.float32),
            pltpu.SemaphoreType.DMA((2,)),
            pltpu.SemaphoreType.DMA((2,)),
        ],
        compiler_params=pltpu.CompilerParams(collective_id=1),
    )(ctx, wo)


def kernel(x, Wq, K_ext, V_ext, Wo):
    k16, v16 = _to_bf16(K_ext, V_ext)
    ctx = _fused_attn(x, Wq, k16, v16)
    return _wo_allreduce(ctx, Wo)


# device time: 309109 ns/iter; 2.5141x vs baseline; 1.2243x over previous
import jax
import jax.numpy as jnp
from jax import lax
from jax.experimental import pallas as pl
from jax.experimental.pallas import tpu as pltpu

N_DEV = 4
SQ = 2048
SKV = 2048
HQ = 8
DH = 128
DM = 1024
SCALE = 0.08838834764831843
MASK_BLK = 64
KCHUNK = 1024
NCHUNK = SKV // KCHUNK
RCH = SQ // N_DEV

_MESH = pl.DeviceIdType.MESH


def _to_bf16(k_ext, v_ext):

    def body(k_ref, v_ref, ko_ref, vo_ref):
        ko_ref[...] = k_ref[...].astype(jnp.bfloat16)
        vo_ref[...] = v_ref[...].astype(jnp.bfloat16)

    n = 16
    blk = pl.BlockSpec((1, SKV // n, 4 * HQ, DH), lambda i: (0, i, 0, 0))
    return pl.pallas_call(
        body,
        grid=(n,),
        in_specs=[blk, blk],
        out_specs=[blk, blk],
        out_shape=[
            jax.ShapeDtypeStruct((1, SKV, 4 * HQ, DH), jnp.bfloat16),
            jax.ShapeDtypeStruct((1, SKV, 4 * HQ, DH), jnp.bfloat16),
        ],
    )(k_ext, v_ext)


def _fused_attn(x, wq, k_ext, v_ext):

    def body(x_ref, wq_ref, k_hbm, v_hbm, o_ref, kbuf, vbuf, relay,
             local_sems, send_sems, recv_sems, relay_recv, fwd_sems,
             credit_sem, ack_sem, p2credit_sem):
        my = lax.axis_index("i")
        barrier = pltpu.get_barrier_semaphore()

        def kchunk(buf, c):
            return buf.at[pl.ds(KCHUNK * c, KCHUNK)]

        @pl.when(my == 0)
        def _():
            for p in (1, 2, 3):
                pl.semaphore_signal(barrier, inc=1, device_id=(p,),
                                    device_id_type=_MESH)
            pl.semaphore_wait(barrier, 3)
            ck = pltpu.make_async_copy(
                k_hbm.at[0, :, pl.ds(0, HQ), :], kbuf, local_sems.at[0])
            cv = pltpu.make_async_copy(
                v_hbm.at[0, :, pl.ds(0, HQ), :], vbuf, local_sems.at[1])
            ck.start()
            cv.start()
            for c in range(NCHUNK):
                b = c * 6
                sends = [
                    (v_hbm, 2 * HQ, relay, relay_recv.at[c], 1),
                    (k_hbm, 2 * HQ, relay, relay_recv.at[c], 3),
                    (k_hbm, 1 * HQ, kbuf, recv_sems.at[c], 1),
                    (v_hbm, 1 * HQ, vbuf, recv_sems.at[NCHUNK + c], 1),
                    (k_hbm, 3 * HQ, kbuf, recv_sems.at[c], 3),
                    (v_hbm, 3 * HQ, vbuf, recv_sems.at[NCHUNK + c], 3),
                ]
                for j, (src, h0, dst, rsem, peer) in enumerate(sends):
                    r = pltpu.make_async_remote_copy(
                        src_ref=src.at[0, pl.ds(KCHUNK * c, KCHUNK),
                                       pl.ds(h0, HQ), :],
                        dst_ref=kchunk(dst, c),
                        send_sem=send_sems.at[b + j],
                        recv_sem=rsem,
                        device_id=(peer,), device_id_type=_MESH)
                    r.start()
            ck.wait()
            cv.wait()

        @pl.when(my != 0)
        def _():
            pl.semaphore_signal(barrier, inc=1, device_id=(0,),
                                device_id_type=_MESH)
            pl.semaphore_wait(barrier, 1)

        q_all = jnp.dot(x_ref[0].astype(jnp.bfloat16),
                        wq_ref[:, :].astype(jnp.bfloat16),
                        preferred_element_type=jnp.float32)

        for c in range(NCHUNK):
            @pl.when(my == 1)
            def _(c=c):
                pltpu.make_async_remote_copy(
                    src_ref=kchunk(relay, c), dst_ref=kchunk(relay, c),
                    send_sem=fwd_sems.at[c], recv_sem=relay_recv.at[c],
                    device_id=(0,), device_id_type=_MESH).wait_recv()
                pltpu.make_async_remote_copy(
                    src_ref=kchunk(relay, c), dst_ref=kchunk(vbuf, c),
                    send_sem=fwd_sems.at[c],
                    recv_sem=recv_sems.at[NCHUNK + c],
                    device_id=(2,), device_id_type=_MESH).start()

            @pl.when(my == 3)
            def _(c=c):
                pltpu.make_async_remote_copy(
                    src_ref=kchunk(relay, c), dst_ref=kchunk(relay, c),
                    send_sem=fwd_sems.at[c], recv_sem=relay_recv.at[c],
                    device_id=(0,), device_id_type=_MESH).wait_recv()
                pltpu.make_async_remote_copy(
                    src_ref=kchunk(relay, c), dst_ref=kchunk(kbuf, c),
                    send_sem=fwd_sems.at[c],
                    recv_sem=recv_sems.at[c],
                    device_id=(2,), device_id_type=_MESH).start()

            @pl.when(my != 0)
            def _(c=c):
                pltpu.make_async_remote_copy(
                    src_ref=kchunk(kbuf, c), dst_ref=kchunk(kbuf, c),
                    send_sem=send_sems.at[0], recv_sem=recv_sems.at[c],
                    device_id=(0,), device_id_type=_MESH).wait_recv()
                pltpu.make_async_remote_copy(
                    src_ref=kchunk(vbuf, c), dst_ref=kchunk(vbuf, c),
                    send_sem=send_sems.at[1],
                    recv_sem=recv_sems.at[NCHUNK + c],
                    device_id=(0,), device_id_type=_MESH).wait_recv()

            L = KCHUNK * (c + 1)
            for h in range(HQ):
                q_h = q_all[KCHUNK * c:KCHUNK * (c + 1),
                            DH * h:DH * (h + 1)].astype(jnp.bfloat16)
                k_h = kbuf[pl.ds(0, L), h, :]
                v_h = vbuf[pl.ds(0, L), h, :]
                s = lax.dot_general(
                    q_h, k_h, (((1,), (1,)), ((), ())),
                    preferred_element_type=jnp.float32) * SCALE
                r = (KCHUNK * c
                     + lax.broadcasted_iota(jnp.int32, (KCHUNK, L), 0))
                cix = lax.broadcasted_iota(jnp.int32, (KCHUNK, L), 1)
                s = jnp.where((cix // MASK_BLK) <= (r // MASK_BLK), s, -1e9)
                m = jnp.max(s, axis=1, keepdims=True)
                w = jnp.exp(s - m)
                w = (w / jnp.sum(w, axis=1, keepdims=True)).astype(jnp.bfloat16)
                o_ref[pl.ds(KCHUNK * c, KCHUNK), pl.ds(DH * h, DH)] = jnp.dot(
                    w, v_h,
                    preferred_element_type=jnp.float32).astype(jnp.bfloat16)

        @pl.when(my == 0)
        def _():
            for i in range(6 * NCHUNK):
                pltpu.make_async_remote_copy(
                    src_ref=kchunk(kbuf, 0), dst_ref=kchunk(kbuf, 0),
                    send_sem=send_sems.at[i], recv_sem=recv_sems.at[0],
                    device_id=(1,), device_id_type=_MESH).wait_send()
            pl.semaphore_wait(credit_sem, 2)
            for p in (1, 3):
                pl.semaphore_signal(ack_sem, inc=1, device_id=(p,),
                                    device_id_type=_MESH)

        @pl.when(jnp.logical_or(my == 1, my == 3))
        def _():
            for c in range(NCHUNK):
                pltpu.make_async_remote_copy(
                    src_ref=kchunk(relay, c), dst_ref=kchunk(relay, c),
                    send_sem=fwd_sems.at[c], recv_sem=recv_sems.at[0],
                    device_id=(2,), device_id_type=_MESH).wait_send()
            pl.semaphore_signal(credit_sem, inc=1, device_id=(0,),
                                device_id_type=_MESH)
            pl.semaphore_wait(p2credit_sem, 1)
            pl.semaphore_wait(ack_sem, 1)

        @pl.when(my == 2)
        def _():
            for p in (1, 3):
                pl.semaphore_signal(p2credit_sem, inc=1, device_id=(p,),
                                    device_id_type=_MESH)

    return pl.pallas_call(
        body,
        out_shape=jax.ShapeDtypeStruct((SQ, DM), jnp.bfloat16),
        in_specs=[
            pl.BlockSpec(memory_space=pltpu.VMEM),
            pl.BlockSpec(memory_space=pltpu.VMEM),
            pl.BlockSpec(memory_space=pl.ANY),
            pl.BlockSpec(memory_space=pl.ANY),
        ],
        out_specs=pl.BlockSpec(memory_space=pltpu.VMEM),
        scratch_shapes=[
            pltpu.VMEM((SKV, HQ, DH), jnp.bfloat16),
            pltpu.VMEM((SKV, HQ, DH), jnp.bfloat16),
            pltpu.VMEM((SKV, HQ, DH), jnp.bfloat16),
            pltpu.SemaphoreType.DMA((2,)),
            pltpu.SemaphoreType.DMA((6 * NCHUNK,)),
            pltpu.SemaphoreType.DMA((2 * NCHUNK,)),
            pltpu.SemaphoreType.DMA((NCHUNK,)),
            pltpu.SemaphoreType.DMA((NCHUNK,)),
            pltpu.SemaphoreType.REGULAR,
            pltpu.SemaphoreType.REGULAR,
            pltpu.SemaphoreType.REGULAR,
        ],
        compiler_params=pltpu.CompilerParams(
            collective_id=0, vmem_limit_bytes=100 * 1024 * 1024),
    )(x, wq, k_ext, v_ext)


def _wo_allreduce(ctx, wo):

    def body(ctx_ref, wo_ref, out_ref, comm, send_sems, recv_sems):
        my = lax.axis_index("i")
        left = (my + N_DEV - 1) % N_DEV
        right = (my + 1) % N_DEV

        barrier = pltpu.get_barrier_semaphore()
        for nbr in (left, right):
            pl.semaphore_signal(barrier, inc=1, device_id=(nbr,),
                                device_id_type=_MESH)
        pl.semaphore_wait(barrier, 2)

        wo16 = wo_ref[:, :].astype(jnp.bfloat16)

        def pchunk(idx):
            off = (idx % N_DEV) * RCH
            return jnp.dot(ctx_ref[pl.ds(off, RCH), :], wo16,
                           preferred_element_type=jnp.float32)

        comm[0] = pchunk(my + N_DEV - 1).astype(jnp.bfloat16)
        acc = None
        for s in range(N_DEV - 1):
            rdma = pltpu.make_async_remote_copy(
                src_ref=comm.at[s % 2],
                dst_ref=comm.at[(s + 1) % 2],
                send_sem=send_sems.at[s % 2],
                recv_sem=recv_sems.at[(s + 1) % 2],
                device_id=(right,), device_id_type=_MESH)
            rdma.start()
            pc = pchunk(my + 2 * N_DEV - 2 - s)
            rdma.wait()
            acc = comm[(s + 1) % 2].astype(jnp.float32) + pc
            comm[(s + 1) % 2] = acc.astype(jnp.bfloat16)

        out_ref[0, pl.ds(my * RCH, RCH), :] = acc

        for t in range(N_DEV - 1):
            src_slot = (1 + t) % 2
            dst_slot = (t % 2)
            rdma = pltpu.make_async_remote_copy(
                src_ref=comm.at[src_slot],
                dst_ref=comm.at[dst_slot],
                send_sem=send_sems.at[src_slot],
                recv_sem=recv_sems.at[dst_slot],
                device_id=(right,), device_id_type=_MESH)
            rdma.start()
            rdma.wait()
            idx = (my + N_DEV - 1 - t) % N_DEV
            out_ref[0, pl.ds(idx * RCH, RCH), :] = (
                comm[dst_slot].astype(jnp.float32))

    return pl.pallas_call(
        body,
        out_shape=jax.ShapeDtypeStruct((1, SQ, DM), jnp.float32),
        in_specs=[
            pl.BlockSpec(memory_space=pltpu.VMEM),
            pl.BlockSpec(memory_space=pltpu.VMEM),
        ],
        out_specs=pl.BlockSpec(memory_space=pltpu.VMEM),
        scratch_shapes=[
            pltpu.VMEM((2, RCH, DM), jnp.bfloat16),
            pltpu.SemaphoreType.DMA((2,)),
            pltpu.SemaphoreType.DMA((2,)),
        ],
        compiler_params=pltpu.CompilerParams(collective_id=1),
    )(ctx, wo)


def kernel(x, Wq, K_ext, V_ext, Wo):
    k16, v16 = _to_bf16(K_ext, V_ext)
    ctx = _fused_attn(x, Wq, k16, v16)
    return _wo_allreduce(ctx, Wo)
